# baseline (speedup 1.0000x reference)
import jax
import jax.numpy as jnp
from jax.experimental import pallas as pl
from jax.experimental.pallas import tpu as pltpu

_BF = jnp.bfloat16
_F32 = jnp.float32
_BLK = 512


def _fused_body(x1_ref, x0_ref, wl_ref, wi_ref, l_ref, it_ref, o_ref,
                a_s, c_s, b_s, d_s, sb_s, sd_s):
    p = pl.program_id(0)
    i = pl.program_id(1)
    rows = pl.ds(i * _BLK, _BLK)

    @pl.when(p == 0)
    def _feature_gemms():
        a_s[rows, :] = jnp.dot(
            x1_ref[...].astype(_BF), wl_ref[...].astype(_BF),
            preferred_element_type=_F32).astype(_BF)
        c_s[rows, :] = jnp.dot(
            x0_ref[...].astype(_BF), wi_ref[...].astype(_BF),
            preferred_element_type=_F32).astype(_BF)

    @pl.when(p == 1)
    def _lap_gemm():
        b = jnp.dot(l_ref[...].astype(_BF), a_s[...],
                    preferred_element_type=_F32)
        b_s[rows, :] = b.astype(_BF)
        csb = jnp.sum(b, axis=0, keepdims=True)

        @pl.when(i == 0)
        def _():
            sb_s[...] = csb

        @pl.when(i > 0)
        def _():
            sb_s[...] += csb

    @pl.when(p == 2)
    def _inc_gemm_and_b_weight():
        d = jnp.dot(it_ref[...].astype(_BF), c_s[...],
                    preferred_element_type=_F32)
        d_s[rows, :] = d.astype(_BF)
        csd = jnp.sum(d, axis=0, keepdims=True)

        @pl.when(i == 0)
        def _():
            sd_s[...] = csd

        @pl.when(i > 0)
        def _():
            sd_s[...] += csd

        sbb = jax.nn.relu(sb_s[...]).astype(_BF).astype(_F32)
        bb = b_s[rows, :].astype(_F32)
        tb = jnp.sum(bb * sbb, axis=1, keepdims=True)
        wb = 1.0 / (1.0 + jnp.exp(-tb))
        b_s[rows, :] = (wb * bb).astype(_BF)

    @pl.when(p == 3)
    def _aggregate():
        sdb = jax.nn.relu(sd_s[...]).astype(_BF).astype(_F32)
        dd = d_s[rows, :].astype(_F32)
        td = jnp.sum(dd * sdb, axis=1, keepdims=True)
        wd = 1.0 / (1.0 + jnp.exp(-td))
        o_ref[...] = jax.nn.relu(
            (b_s[rows, :].astype(_F32) + wd * dd) * 0.5)


def kernel(x0, x1, down_lap_0, incidence_t_0, W_lap, W_inc):
    n1, c = x1.shape
    n0 = x0.shape[0]
    nblk = n1 // _BLK
    last = nblk - 1

    x1_new = pl.pallas_call(
        _fused_body,
        grid=(4, nblk),
        in_specs=[
            pl.BlockSpec((_BLK, c), lambda p, i: (jnp.where(p == 0, i, last), 0)),
            pl.BlockSpec((_BLK, c), lambda p, i: (jnp.where(p == 0, i, last), 0)),
            pl.BlockSpec((c, c), lambda p, i: (0, 0)),
            pl.BlockSpec((c, c), lambda p, i: (0, 0)),
            pl.BlockSpec((_BLK, n1),
                         lambda p, i: (jnp.where(p == 1, i,
                                                 jnp.where(p == 0, 0, last)), 0)),
            pl.BlockSpec((_BLK, n0),
                         lambda p, i: (jnp.where(p == 2, i,
                                                 jnp.where(p < 2, 0, last)), 0)),
        ],
        out_specs=pl.BlockSpec((_BLK, c), lambda p, i: (jnp.where(p == 3, i, 0), 0)),
        out_shape=jax.ShapeDtypeStruct((n1, c), _F32),
        scratch_shapes=[
            pltpu.VMEM((n1, c), _BF),
            pltpu.VMEM((n0, c), _BF),
            pltpu.VMEM((n1, c), _BF),
            pltpu.VMEM((n1, c), _BF),
            pltpu.VMEM((1, c), _F32),
            pltpu.VMEM((1, c), _F32),
        ],
    )(x1, x0, W_lap, W_inc, down_lap_0, incidence_t_0)

    return (x0, x1_new)


# speedup vs baseline: 1.0704x; 1.0704x over previous
import jax
import jax.numpy as jnp
from jax.experimental import pallas as pl
from jax.experimental.pallas import tpu as pltpu

_BF = jnp.bfloat16
_F32 = jnp.float32
_BLK = 512


def _fused_body(x1_ref, x0_ref, wl_ref, wi_ref, l_ref, it_ref, o_ref,
                a_s, c_s, b_s, d_s, sb_s, sd_s):
    p = pl.program_id(0)
    i = pl.program_id(1)
    rows = pl.ds(i * _BLK, _BLK)

    @pl.when(p == 0)
    def _feature_gemms():
        a_s[rows, :] = jnp.dot(
            x1_ref[...].astype(_BF), wl_ref[...].astype(_BF),
            preferred_element_type=_F32).astype(_BF)
        c_s[rows, :] = jnp.dot(
            x0_ref[...].astype(_BF), wi_ref[...].astype(_BF),
            preferred_element_type=_F32).astype(_BF)

    @pl.when(p == 1)
    def _neighborhood_gemms():
        b = jnp.dot(l_ref[...].astype(_BF), a_s[...],
                    preferred_element_type=_F32)
        d = jnp.dot(it_ref[...].astype(_BF), c_s[...],
                    preferred_element_type=_F32)
        b_s[rows, :] = b.astype(_BF)
        d_s[rows, :] = d.astype(_BF)
        csb = jnp.sum(b, axis=0, keepdims=True)
        csd = jnp.sum(d, axis=0, keepdims=True)

        @pl.when(i == 0)
        def _():
            sb_s[...] = csb
            sd_s[...] = csd

        @pl.when(i > 0)
        def _():
            sb_s[...] += csb
            sd_s[...] += csd

    @pl.when(p == 2)
    def _aggregate():
        sbb = jax.nn.relu(sb_s[...]).astype(_BF).astype(_F32)
        sdb = jax.nn.relu(sd_s[...]).astype(_BF).astype(_F32)
        b = b_s[rows, :].astype(_F32)
        d = d_s[rows, :].astype(_F32)
        tb = jnp.sum(b * sbb, axis=1, keepdims=True)
        td = jnp.sum(d * sdb, axis=1, keepdims=True)
        wb = 1.0 / (1.0 + jnp.exp(-tb))
        wd = 1.0 / (1.0 + jnp.exp(-td))
        o_ref[...] = jax.nn.relu((wb * b + wd * d) * 0.5)


def kernel(x0, x1, down_lap_0, incidence_t_0, W_lap, W_inc):
    n1, c = x1.shape
    n0 = x0.shape[0]
    nblk = n1 // _BLK
    last = nblk - 1

    x1_new = pl.pallas_call(
        _fused_body,
        grid=(3, nblk),
        in_specs=[
            pl.BlockSpec((_BLK, c), lambda p, i: (jnp.where(p == 0, i, last), 0)),
            pl.BlockSpec((_BLK, c), lambda p, i: (jnp.where(p == 0, i, last), 0)),
            pl.BlockSpec((c, c), lambda p, i: (0, 0)),
            pl.BlockSpec((c, c), lambda p, i: (0, 0)),
            pl.BlockSpec((_BLK, n1),
                         lambda p, i: (jnp.where(p == 1, i,
                                                 jnp.where(p == 0, 0, last)), 0)),
            pl.BlockSpec((_BLK, n0),
                         lambda p, i: (jnp.where(p == 1, i,
                                                 jnp.where(p == 0, 0, last)), 0)),
        ],
        out_specs=pl.BlockSpec((_BLK, c), lambda p, i: (jnp.where(p == 2, i, 0), 0)),
        out_shape=jax.ShapeDtypeStruct((n1, c), _F32),
        scratch_shapes=[
            pltpu.VMEM((n1, c), _BF),
            pltpu.VMEM((n0, c), _BF),
            pltpu.VMEM((n1, c), _BF),
            pltpu.VMEM((n1, c), _BF),
            pltpu.VMEM((1, c), _F32),
            pltpu.VMEM((1, c), _F32),
        ],
    )(x1, x0, W_lap, W_inc, down_lap_0, incidence_t_0)

    return (x0, x1_new)
